# baseline (speedup 1.0000x reference)
import functools

import numpy as np

import jax
import jax.numpy as jnp
from jax import lax
from jax.experimental import pallas as pl
from jax.experimental.pallas import tpu as pltpu
from jax.experimental.pallas import tpu_sc as plsc

N = 10000
E = 320000
NC, NS, L = 2, 16, 16
NW = NC * NS
C = 128
CH = 79
EPW = C * CH
E_PAD = NW * EPW
N_PAD = 79 * 128
NPW = N_PAD // NS

_mesh = functools.partial(
    plsc.VectorSubcoreMesh,
    core_axis_name="c", subcore_axis_name="s",
    num_cores=NC, num_subcores=NS)


def _rsqrt16(x):
    b = lax.bitcast_convert_type(x, jnp.int32)
    i = jnp.int32(0x5F3759DF) - lax.shift_right_logical(b, 1)
    y = lax.bitcast_convert_type(i, jnp.float32)
    for _ in range(4):
        y = y * (1.5 - 0.5 * x * y * y)
    return y


def _zero_rows(row_v, d, n):
    def body(i, carry):
        for k in range(d // L):
            row_v[i, pl.ds(k * L, L)] = jnp.zeros((L,), jnp.float32)
        return carry
    lax.fori_loop(0, n, body, 0)


CC = 64
NCH = EPW // CC
NCHP = 160


def _agg_phase(h_hbm, out0_hbm, out1_hbm, packed_v, nrm_v, srcring, dstring,
               rowbf, rowf, gsems, ssems, acc_sh, c, s, d):
    mask16 = jnp.int32(0xFFFF)

    def unpack(jj, slot):
        for g in range(CC // L):
            sl = pl.ds(g * L, L)
            p = packed_v[jj, sl]
            srcring[slot, sl] = lax.bitwise_and(p, mask16)
            dstring[slot, sl] = lax.shift_right_logical(p, 16)

    def g_issue(b3):
        pltpu.async_copy(h_hbm.at[srcring.at[b3]], rowbf[b3], gsems[b3])

    def g_wait(b3):
        pltpu.make_async_copy(h_hbm.at[srcring.at[b3]], rowbf[b3],
                              gsems[b3]).wait()

    def s_issue(b2, b3):
        pltpu.async_copy(rowf[b2], acc_sh.at[dstring.at[b3]],
                         ssems[b2], add=True)

    def s_wait(b2, b3):
        pltpu.make_async_copy(rowf[b2], acc_sh.at[dstring.at[b3]],
                              ssems[b2]).wait()

    def scale(j, b3, b2):
        jj = jnp.full((L,), j, dtype=jnp.int32)
        rb = rowbf[b3]
        rf = rowf[b2]

        def edge8(q, ecarry):
            for t in range(8):
                e = q * 8 + t
                w = plsc.load_gather(nrm_v, [jj, jnp.full((L,), e, jnp.int32)])
                for k in range(d // 32):
                    wi = rb[e, pl.ds(k * L, L)]
                    ev = lax.bitcast_convert_type(
                        lax.shift_left(wi, 16), jnp.float32)
                    od = lax.bitcast_convert_type(
                        lax.bitwise_and(wi, jnp.int32(-65536)), jnp.float32)
                    rf[e, pl.ds(k * 32, L)] = ev * w
                    rf[e, pl.ds(k * 32 + L, L)] = od * w
            return ecarry
        lax.fori_loop(0, CC // 8, edge8, 0)

    def turn(j, b3, b2):
        bp3 = (b3 + 2) % 3
        b2p = 1 - b2
        g_wait(b3)
        scale(j, b3, b2)
        s_issue(b2, b3)

        @pl.when(j >= 1)
        def _():
            s_wait(b2p, bp3)
        unpack(j + 2, bp3)

        @pl.when(j + 2 < NCH)
        def _():
            g_issue(bp3)

    unpack(jnp.int32(0), 0)
    unpack(jnp.int32(1), 1)
    g_issue(0)
    g_issue(1)

    _zero_rows(rowf[0], d, CC)
    base = s * NPW
    for t in range(NPW // CC):
        pltpu.sync_copy(rowf[0], acc_sh.at[pl.ds(base + t * CC, CC)])
    rem = NPW % CC
    if rem:
        pltpu.sync_copy(rowf[0].at[pl.ds(0, rem)],
                        acc_sh.at[pl.ds(base + (NPW // CC) * CC, rem)])
    plsc.subcore_barrier()

    def body(g, carry):
        j = g * 6
        turn(j, 0, 0)
        turn(j + 1, 1, 1)
        turn(j + 2, 2, 0)
        turn(j + 3, 0, 1)
        turn(j + 4, 1, 0)
        turn(j + 5, 2, 1)
        return carry
    lax.fori_loop(0, NCH // 6, body, 0)
    turn(jnp.int32(NCH - 2), (NCH - 2) % 3, (NCH - 2) % 2)
    turn(jnp.int32(NCH - 1), (NCH - 1) % 3, (NCH - 1) % 2)
    s_wait((NCH - 1) % 2, (NCH - 1) % 3)

    plsc.subcore_barrier()

    @pl.when(c == 0)
    def _():
        pltpu.sync_copy(acc_sh.at[pl.ds(s * NPW, NPW)],
                        out0_hbm.at[pl.ds(s * NPW, NPW)])

    @pl.when(c == 1)
    def _():
        pltpu.sync_copy(acc_sh.at[pl.ds(s * NPW, NPW)],
                        out1_hbm.at[pl.ds(s * NPW, NPW)])


def _off640(s):
    return jnp.minimum(s * 640, N_PAD - 640)


def _dump_partials(acc_sh, out0_hbm, out1_hbm, c, s):
    off = _off640(s)

    @pl.when(c == 0)
    def _():
        pltpu.sync_copy(acc_sh.at[pl.ds(off, 640)],
                        out0_hbm.at[pl.ds(off, 640)])

    @pl.when(c == 1)
    def _():
        pltpu.sync_copy(acc_sh.at[pl.ds(off, 640)],
                        out1_hbm.at[pl.ds(off, 640)])


def _sc_deg(ew3, dst3):
    @functools.partial(
        pl.kernel,
        out_type=(jax.ShapeDtypeStruct((N_PAD,), jnp.float32),
                  jax.ShapeDtypeStruct((N_PAD,), jnp.float32)),
        mesh=_mesh(),
        compiler_params=pltpu.CompilerParams(needs_layout_passes=False, use_tc_tiling_on_sc=False),
        scratch_types=[
            pltpu.VMEM((CH, C), jnp.float32),
            pltpu.VMEM((CH, C), jnp.int32),
            pltpu.VMEM((640,), jnp.float32),
            pltpu.VMEM_SHARED((N_PAD,), jnp.float32),
        ])
    def k(ew_hbm, dst_hbm, out0_hbm, out1_hbm, ew_v, dst_v, zro_v, acc_sh):
        c = lax.axis_index("c")
        s = lax.axis_index("s")
        wid = c * NS + s
        pltpu.sync_copy(ew_hbm.at[wid], ew_v)
        pltpu.sync_copy(dst_hbm.at[wid], dst_v)

        def z(i, carry):
            zro_v[pl.ds(i * L, L)] = jnp.zeros((L,), jnp.float32)
            return carry
        lax.fori_loop(0, 640 // L, z, 0)
        pltpu.sync_copy(zro_v, acc_sh.at[pl.ds(_off640(s), 640)])
        plsc.subcore_barrier()

        def chunk(j, carry):
            pltpu.sync_copy(ew_v.at[j], acc_sh.at[dst_v.at[j]], add=True)
            return carry
        lax.fori_loop(0, CH, chunk, 0)
        plsc.subcore_barrier()
        _dump_partials(acc_sh, out0_hbm, out1_hbm, c, s)

    return k(ew3, dst3)


def _sc_norm(src3, dst3, ew3, deg0, deg1):
    @functools.partial(
        pl.kernel,
        out_type=(jax.ShapeDtypeStruct((NW, CH, C), jnp.float32),
                  jax.ShapeDtypeStruct((N_PAD,), jnp.float32)),
        mesh=_mesh(),
        compiler_params=pltpu.CompilerParams(needs_layout_passes=False, use_tc_tiling_on_sc=False),
        scratch_types=[
            pltpu.VMEM((CH, C), jnp.int32),
            pltpu.VMEM((CH, C), jnp.int32),
            pltpu.VMEM((CH, C), jnp.float32),
            pltpu.VMEM((CH, C), jnp.float32),
            pltpu.VMEM((N_PAD,), jnp.float32),
            pltpu.VMEM((N_PAD,), jnp.float32),
        ])
    def k(src_hbm, dst_hbm, ew_hbm, deg0_hbm, deg1_hbm,
          nrm_hbm, invdeg_hbm,
          src_v, dst_v, ew_v, nrm_v, p0_v, p1_v):
        c = lax.axis_index("c")
        s = lax.axis_index("s")
        wid = c * NS + s
        pltpu.sync_copy(src_hbm.at[wid], src_v)
        pltpu.sync_copy(dst_hbm.at[wid], dst_v)
        pltpu.sync_copy(ew_hbm.at[wid], ew_v)
        pltpu.sync_copy(deg0_hbm, p0_v)
        pltpu.sync_copy(deg1_hbm, p1_v)

        def dv(i, carry):
            sl = pl.ds(i * L, L)
            d = p0_v[sl] + p1_v[sl] + 1.0
            y = _rsqrt16(d)
            p1_v[sl] = y
            p0_v[sl] = y * y
            return carry
        lax.fori_loop(0, N_PAD // L, dv, 0)

        @pl.when(c == 0)
        def _():
            off = _off640(s)
            pltpu.sync_copy(p0_v.at[pl.ds(off, 640)],
                            invdeg_hbm.at[pl.ds(off, 640)])

        def nj(j, carry):
            for g in range(C // L):
                sl = pl.ds(g * L, L)
                nv = (plsc.load_gather(p1_v, [src_v[j, sl]])
                      * ew_v[j, sl]
                      * plsc.load_gather(p1_v, [dst_v[j, sl]]))
                nrm_v[j, sl] = nv
            return carry
        lax.fori_loop(0, CH, nj, 0)
        pltpu.sync_copy(nrm_v, nrm_hbm.at[wid])

    return k(src3, dst3, ew3, deg0, deg1)


def _sc_agg(hbf, packed3, nrm3, d):
    @functools.partial(
        pl.kernel,
        out_type=(jax.ShapeDtypeStruct((N_PAD, d), jnp.float32),
                  jax.ShapeDtypeStruct((N_PAD, d), jnp.float32)),
        mesh=_mesh(),
        compiler_params=pltpu.CompilerParams(needs_layout_passes=False, use_tc_tiling_on_sc=False),
        scratch_types=[
            pltpu.VMEM((NCHP, CC), jnp.int32),
            pltpu.VMEM((NCH, CC), jnp.float32),
            pltpu.VMEM((3, CC), jnp.int32),
            pltpu.VMEM((3, CC), jnp.int32),
            pltpu.VMEM((CC, d // 2), jnp.int32),
            pltpu.VMEM((CC, d // 2), jnp.int32),
            pltpu.VMEM((CC, d // 2), jnp.int32),
            pltpu.VMEM((CC, d), jnp.float32),
            pltpu.VMEM((CC, d), jnp.float32),
            pltpu.VMEM_SHARED((N_PAD, d), jnp.float32),
            pltpu.SemaphoreType.DMA,
            pltpu.SemaphoreType.DMA,
            pltpu.SemaphoreType.DMA,
            pltpu.SemaphoreType.DMA,
            pltpu.SemaphoreType.DMA,
        ])
    def k(h_hbm, packed_hbm, nrm_hbm, out0_hbm, out1_hbm,
          packed_v, nrm_v, srcring, dstring, rb0, rb1, rb2, rf0, rf1, acc_sh,
          g0, g1, g2, s0, s1):
        c = lax.axis_index("c")
        s = lax.axis_index("s")
        wid = c * NS + s
        pltpu.sync_copy(packed_hbm.at[wid], packed_v)
        pltpu.sync_copy(nrm_hbm.at[wid], nrm_v)
        _agg_phase(h_hbm, out0_hbm, out1_hbm, packed_v, nrm_v, srcring,
                   dstring, (rb0, rb1, rb2), (rf0, rf1), (g0, g1, g2),
                   (s0, s1), acc_sh, c, s, d)

    return k(hbf, packed3, nrm3)


def _sc_layer3(h3, src3, dst3, nrm3):
    @functools.partial(
        pl.kernel,
        out_type=(jax.ShapeDtypeStruct((N_PAD,), jnp.float32),
                  jax.ShapeDtypeStruct((N_PAD,), jnp.float32)),
        mesh=_mesh(),
        compiler_params=pltpu.CompilerParams(needs_layout_passes=False, use_tc_tiling_on_sc=False),
        scratch_types=[
            pltpu.VMEM((N_PAD,), jnp.float32),
            pltpu.VMEM((CH, C), jnp.int32),
            pltpu.VMEM((CH, C), jnp.int32),
            pltpu.VMEM((CH, C), jnp.float32),
            pltpu.VMEM((C,), jnp.float32),
            pltpu.VMEM((640,), jnp.float32),
            pltpu.VMEM_SHARED((N_PAD,), jnp.float32),
        ])
    def k(h_hbm, src_hbm, dst_hbm, nrm_hbm, out0_hbm, out1_hbm,
          tab_v, src_v, dst_v, nrm_v, val_v, zro_v, acc_sh):
        c = lax.axis_index("c")
        s = lax.axis_index("s")
        wid = c * NS + s
        pltpu.sync_copy(h_hbm, tab_v)
        pltpu.sync_copy(src_hbm.at[wid], src_v)
        pltpu.sync_copy(dst_hbm.at[wid], dst_v)
        pltpu.sync_copy(nrm_hbm.at[wid], nrm_v)

        def z(i, carry):
            zro_v[pl.ds(i * L, L)] = jnp.zeros((L,), jnp.float32)
            return carry
        lax.fori_loop(0, 640 // L, z, 0)
        pltpu.sync_copy(zro_v, acc_sh.at[pl.ds(_off640(s), 640)])
        plsc.subcore_barrier()

        def chunk(j, carry):
            for g in range(C // L):
                sl = pl.ds(g * L, L)
                v = plsc.load_gather(tab_v, [src_v[j, sl]]) * nrm_v[j, sl]
                val_v[sl] = v
            pltpu.sync_copy(val_v, acc_sh.at[dst_v.at[j]], add=True)
            return carry
        lax.fori_loop(0, CH, chunk, 0)
        plsc.subcore_barrier()
        _dump_partials(acc_sh, out0_hbm, out1_hbm, c, s)

    return k(h3, src3, dst3, nrm3)


def _ew_body(d_ref, o_ref):
    d = d_ref[...] * (1.0 / 200.0)
    o_ref[...] = jnp.exp(-(d * d))


def _tc_ew(d2):
    return pl.pallas_call(
        _ew_body,
        grid=(4,),
        in_specs=[pl.BlockSpec((E_PAD // 128 // 4, 128), lambda i: (i, 0))],
        out_specs=pl.BlockSpec((E_PAD // 128 // 4, 128), lambda i: (i, 0)),
        out_shape=jax.ShapeDtypeStruct((E_PAD // 128, 128), jnp.float32),
    )(d2)


def _half_perms(d):
    lo = np.concatenate([np.arange(k, k + 16) for k in range(0, d, 32)])
    hi = lo + 16
    mk = lambda q: jnp.asarray(
        (np.arange(d)[:, None] == q[None, :]).astype(np.float32))
    return mk(lo), mk(hi)


def _pack_rows(h, plo_ref, phi_ref):
    lo = jnp.dot(h, plo_ref[...], precision=lax.Precision.HIGHEST,
                 preferred_element_type=jnp.float32)
    hi = jnp.dot(h, phi_ref[...], precision=lax.Precision.HIGHEST,
                 preferred_element_type=jnp.float32)
    bl = lax.shift_right_logical(
        lax.bitcast_convert_type(lo.astype(jnp.bfloat16).astype(jnp.float32),
                                 jnp.int32), 16)
    bh = lax.bitwise_and(
        lax.bitcast_convert_type(hi.astype(jnp.bfloat16).astype(jnp.float32),
                                 jnp.int32), jnp.int32(-65536))
    return lax.bitwise_or(bl, bh)


def _mm_body(x_ref, w_ref, plo_ref, phi_ref, o_ref, t_ref):
    h = jnp.dot(x_ref[...], w_ref[...], precision=lax.Precision.HIGHEST,
                preferred_element_type=jnp.float32)
    o_ref[...] = h
    t_ref[...] = _pack_rows(h, plo_ref, phi_ref)


def _tc_matmul(xp, w, plo, phi):
    m, kdim = xp.shape
    n = w.shape[1]
    return pl.pallas_call(
        _mm_body,
        grid=(m // 128,),
        in_specs=[pl.BlockSpec((128, kdim), lambda i: (i, 0)),
                  pl.BlockSpec((kdim, n), lambda i: (0, 0)),
                  pl.BlockSpec((n, n // 2), lambda i: (0, 0)),
                  pl.BlockSpec((n, n // 2), lambda i: (0, 0))],
        out_specs=[pl.BlockSpec((128, n), lambda i: (i, 0)),
                   pl.BlockSpec((128, n // 2), lambda i: (i, 0))],
        out_shape=[jax.ShapeDtypeStruct((m, n), jnp.float32),
                   jax.ShapeDtypeStruct((m, n // 2), jnp.int32)],
    )(xp, w, plo, phi)


def _fuse1_body(a0_ref, a1_ref, h_ref, iv_ref, b_ref, w_ref, plo_ref,
                phi_ref, o_ref, t_ref):
    a = a0_ref[...] + a1_ref[...] + h_ref[...] * iv_ref[...] + b_ref[...]
    h2 = jnp.dot(jnp.maximum(a, 0.0), w_ref[...],
                 precision=lax.Precision.HIGHEST,
                 preferred_element_type=jnp.float32)
    o_ref[...] = h2
    t_ref[...] = _pack_rows(h2, plo_ref, phi_ref)


def _tc_fuse1(a0, a1, h1, invd, b1, w2, plo, phi):
    return pl.pallas_call(
        _fuse1_body,
        grid=(N_PAD // 128,),
        in_specs=[pl.BlockSpec((128, 128), lambda i: (i, 0)),
                  pl.BlockSpec((128, 128), lambda i: (i, 0)),
                  pl.BlockSpec((128, 128), lambda i: (i, 0)),
                  pl.BlockSpec((128, 1), lambda i: (i, 0)),
                  pl.BlockSpec((1, 128), lambda i: (0, 0)),
                  pl.BlockSpec((128, 64), lambda i: (0, 0)),
                  pl.BlockSpec((64, 32), lambda i: (0, 0)),
                  pl.BlockSpec((64, 32), lambda i: (0, 0))],
        out_specs=[pl.BlockSpec((128, 64), lambda i: (i, 0)),
                   pl.BlockSpec((128, 32), lambda i: (i, 0))],
        out_shape=[jax.ShapeDtypeStruct((N_PAD, 64), jnp.float32),
                   jax.ShapeDtypeStruct((N_PAD, 32), jnp.int32)],
    )(a0, a1, h1, invd, b1.reshape(1, 128), w2, plo, phi)


def _fuse2_body(a0_ref, a1_ref, h_ref, iv_ref, b_ref, w_ref, o_ref):
    a = a0_ref[...] + a1_ref[...] + h_ref[...] * iv_ref[...] + b_ref[...]
    o_ref[...] = jnp.sum(jnp.maximum(a, 0.0) * w_ref[...], axis=1,
                         keepdims=True)


def _tc_fuse2(a0, a1, h2, invd, b2, w3):
    return pl.pallas_call(
        _fuse2_body,
        grid=(N_PAD // 128,),
        in_specs=[pl.BlockSpec((128, 64), lambda i: (i, 0)),
                  pl.BlockSpec((128, 64), lambda i: (i, 0)),
                  pl.BlockSpec((128, 64), lambda i: (i, 0)),
                  pl.BlockSpec((128, 1), lambda i: (i, 0)),
                  pl.BlockSpec((1, 64), lambda i: (0, 0)),
                  pl.BlockSpec((1, 64), lambda i: (0, 0))],
        out_specs=pl.BlockSpec((128, 1), lambda i: (i, 0)),
        out_shape=jax.ShapeDtypeStruct((N_PAD, 1), jnp.float32),
    )(a0, a1, h2, invd, b2.reshape(1, 64), w3.reshape(1, 64))


def _fuse3_body(a0_ref, a1_ref, h_ref, iv_ref, b_ref, o_ref):
    o_ref[...] = (a0_ref[...] + a1_ref[...] + h_ref[...] * iv_ref[...]
                  + b_ref[...])


def _tc_fuse3(a0, a1, h3r, invr, b3):
    return pl.pallas_call(
        _fuse3_body,
        grid=(1,),
        in_specs=[pl.BlockSpec((CH, 128), lambda i: (0, 0)),
                  pl.BlockSpec((CH, 128), lambda i: (0, 0)),
                  pl.BlockSpec((CH, 128), lambda i: (0, 0)),
                  pl.BlockSpec((CH, 128), lambda i: (0, 0)),
                  pl.BlockSpec((1, 1), lambda i: (0, 0))],
        out_specs=pl.BlockSpec((CH, 128), lambda i: (0, 0)),
        out_shape=jax.ShapeDtypeStruct((CH, 128), jnp.float32),
    )(a0, a1, h3r, invr, b3.reshape(1, 1))


def kernel(x, edge_index, edge_attr, W1, b1, W2, b2, W3, b3):
    src = edge_index[0].astype(jnp.int32)
    dst = edge_index[1].astype(jnp.int32)
    src3 = jnp.pad(src, (0, E_PAD - E)).reshape(NW, CH, C)
    dst3 = jnp.pad(dst, (0, E_PAD - E)).reshape(NW, CH, C)
    dpad = jnp.pad(edge_attr.reshape(-1), (0, E_PAD - E),
                   constant_values=1e9)
    xp = jnp.pad(x, ((0, N_PAD - N), (0, 0)))

    ew2 = _tc_ew(dpad.reshape(E_PAD // 128, 128))
    ew3 = ew2.reshape(NW, CH, C)
    plo128, phi128 = _half_perms(128)
    plo64, phi64 = _half_perms(64)
    h1, t1 = _tc_matmul(xp, W1, plo128, phi128)

    deg0, deg1 = _sc_deg(ew3, dst3)
    nrm, invdeg = _sc_norm(src3, dst3, ew3, deg0, deg1)
    invd2 = invdeg.reshape(N_PAD, 1)

    packed = src3.reshape(NW, EPW) | (dst3.reshape(NW, EPW) << 16)
    packed3 = jnp.pad(packed, ((0, 0), (0, NCHP * CC - EPW))
                      ).reshape(NW, NCHP, CC)
    nrm158 = nrm.reshape(NW, NCH, CC)

    a10, a11 = _sc_agg(t1, packed3, nrm158, 128)
    h2, t2 = _tc_fuse1(a10, a11, h1, invd2, b1, W2, plo64, phi64)
    a20, a21 = _sc_agg(t2, packed3, nrm158, 64)
    h3 = _tc_fuse2(a20, a21, h2, invd2, b2, W3)
    a30, a31 = _sc_layer3(h3.reshape(-1), src3, dst3, nrm)

    out = _tc_fuse3(a30.reshape(CH, 128), a31.reshape(CH, 128),
                    h3.reshape(CH, 128),
                    invdeg.reshape(CH, 128), b3)
    return out.reshape(-1)[:N]


# speedup vs baseline: 1.0019x; 1.0019x over previous
import functools

import numpy as np

import jax
import jax.numpy as jnp
from jax import lax
from jax.experimental import pallas as pl
from jax.experimental.pallas import tpu as pltpu
from jax.experimental.pallas import tpu_sc as plsc

N = 10000
E = 320000
NC, NS, L = 2, 16, 16
NW = NC * NS
C = 128
CH = 79
EPW = C * CH
E_PAD = NW * EPW
N_PAD = 79 * 128
NPW = N_PAD // NS

_mesh = functools.partial(
    plsc.VectorSubcoreMesh,
    core_axis_name="c", subcore_axis_name="s",
    num_cores=NC, num_subcores=NS)


def _rsqrt16(x):
    b = lax.bitcast_convert_type(x, jnp.int32)
    i = jnp.int32(0x5F3759DF) - lax.shift_right_logical(b, 1)
    y = lax.bitcast_convert_type(i, jnp.float32)
    for _ in range(4):
        y = y * (1.5 - 0.5 * x * y * y)
    return y


def _zero_rows(row_v, d, n):
    def body(i, carry):
        for k in range(d // L):
            row_v[i, pl.ds(k * L, L)] = jnp.zeros((L,), jnp.float32)
        return carry
    lax.fori_loop(0, n, body, 0)


CC = 64
NCH = EPW // CC
NCHP = 160


def _agg_phase(h_hbm, out0_hbm, out1_hbm, packed_v, nrm_v, srcring, dstring,
               rowbf, rowf, gsems, ssems, acc_sh, c, s, d):
    mask16 = jnp.int32(0xFFFF)

    def unpack(jj, slot):
        for g in range(CC // L):
            sl = pl.ds(g * L, L)
            p = packed_v[jj, sl]
            srcring[slot, sl] = lax.bitwise_and(p, mask16)
            dstring[slot, sl] = lax.shift_right_logical(p, 16)

    def g_issue(b3):
        pltpu.async_copy(h_hbm.at[srcring.at[b3]], rowbf[b3], gsems[b3])

    def g_wait(b3):
        pltpu.make_async_copy(h_hbm.at[srcring.at[b3]], rowbf[b3],
                              gsems[b3]).wait()

    def s_issue(b2, b3):
        pltpu.async_copy(rowf[b2], acc_sh.at[dstring.at[b3]],
                         ssems[b2], add=True)

    def s_wait(b2, b3):
        pltpu.make_async_copy(rowf[b2], acc_sh.at[dstring.at[b3]],
                              ssems[b2]).wait()

    def scale(j, b3, b2):
        jj = jnp.full((L,), j, dtype=jnp.int32)
        rb = rowbf[b3]
        rf = rowf[b2]

        def edge8(q, ecarry):
            for t in range(8):
                e = q * 8 + t
                w = plsc.load_gather(nrm_v, [jj, jnp.full((L,), e, jnp.int32)])
                for k in range(d // 32):
                    wi = rb[e, pl.ds(k * L, L)]
                    ev = lax.bitcast_convert_type(
                        lax.shift_left(wi, 16), jnp.float32)
                    od = lax.bitcast_convert_type(
                        lax.bitwise_and(wi, jnp.int32(-65536)), jnp.float32)
                    rf[e, pl.ds(k * 32, L)] = ev * w
                    rf[e, pl.ds(k * 32 + L, L)] = od * w
            return ecarry
        lax.fori_loop(0, CC // 8, edge8, 0)

    def turn(j, b3, b2):
        bp3 = (b3 + 2) % 3
        b2p = 1 - b2
        g_wait(b3)
        scale(j, b3, b2)
        s_issue(b2, b3)

        @pl.when(j >= 1)
        def _():
            s_wait(b2p, bp3)
        unpack(j + 2, bp3)

        @pl.when(j + 2 < NCH)
        def _():
            g_issue(bp3)

    unpack(jnp.int32(0), 0)
    unpack(jnp.int32(1), 1)
    g_issue(0)
    g_issue(1)

    _zero_rows(rowf[0], d, CC)
    base = s * NPW
    for t in range(NPW // CC):
        pltpu.sync_copy(rowf[0], acc_sh.at[pl.ds(base + t * CC, CC)])
    rem = NPW % CC
    if rem:
        pltpu.sync_copy(rowf[0].at[pl.ds(0, rem)],
                        acc_sh.at[pl.ds(base + (NPW // CC) * CC, rem)])
    plsc.subcore_barrier()

    def body(g, carry):
        j = g * 6
        turn(j, 0, 0)
        turn(j + 1, 1, 1)
        turn(j + 2, 2, 0)
        turn(j + 3, 0, 1)
        turn(j + 4, 1, 0)
        turn(j + 5, 2, 1)
        return carry
    lax.fori_loop(0, NCH // 6, body, 0)
    turn(jnp.int32(NCH - 2), (NCH - 2) % 3, (NCH - 2) % 2)
    turn(jnp.int32(NCH - 1), (NCH - 1) % 3, (NCH - 1) % 2)
    s_wait((NCH - 1) % 2, (NCH - 1) % 3)

    plsc.subcore_barrier()

    @pl.when(c == 0)
    def _():
        pltpu.sync_copy(acc_sh.at[pl.ds(s * NPW, NPW)],
                        out0_hbm.at[pl.ds(s * NPW, NPW)])

    @pl.when(c == 1)
    def _():
        pltpu.sync_copy(acc_sh.at[pl.ds(s * NPW, NPW)],
                        out1_hbm.at[pl.ds(s * NPW, NPW)])


def _off640(s):
    return jnp.minimum(s * 640, N_PAD - 640)


def _dump_partials(acc_sh, out0_hbm, out1_hbm, c, s):
    off = _off640(s)

    @pl.when(c == 0)
    def _():
        pltpu.sync_copy(acc_sh.at[pl.ds(off, 640)],
                        out0_hbm.at[pl.ds(off, 640)])

    @pl.when(c == 1)
    def _():
        pltpu.sync_copy(acc_sh.at[pl.ds(off, 640)],
                        out1_hbm.at[pl.ds(off, 640)])


def _ew16(d):
    t = d * (1.0 / 200.0)
    return jnp.exp(-(t * t))


def _sc_deg(d3, dst3):
    @functools.partial(
        pl.kernel,
        out_type=(jax.ShapeDtypeStruct((N_PAD,), jnp.float32),
                  jax.ShapeDtypeStruct((N_PAD,), jnp.float32)),
        mesh=_mesh(),
        compiler_params=pltpu.CompilerParams(needs_layout_passes=False, use_tc_tiling_on_sc=False),
        scratch_types=[
            pltpu.VMEM((CH, C), jnp.float32),
            pltpu.VMEM((CH, C), jnp.int32),
            pltpu.VMEM((640,), jnp.float32),
            pltpu.VMEM_SHARED((N_PAD,), jnp.float32),
        ])
    def k(d_hbm, dst_hbm, out0_hbm, out1_hbm, ew_v, dst_v, zro_v, acc_sh):
        c = lax.axis_index("c")
        s = lax.axis_index("s")
        wid = c * NS + s
        pltpu.sync_copy(d_hbm.at[wid], ew_v)
        pltpu.sync_copy(dst_hbm.at[wid], dst_v)

        def ew_all(i, carry):
            r = i // (C // L)
            g = i % (C // L)
            sl = pl.ds(g * L, L)
            ew_v[r, sl] = _ew16(ew_v[r, sl])
            return carry
        lax.fori_loop(0, CH * (C // L), ew_all, 0)

        def z(i, carry):
            zro_v[pl.ds(i * L, L)] = jnp.zeros((L,), jnp.float32)
            return carry
        lax.fori_loop(0, 640 // L, z, 0)
        pltpu.sync_copy(zro_v, acc_sh.at[pl.ds(_off640(s), 640)])
        plsc.subcore_barrier()

        def chunk(j, carry):
            pltpu.sync_copy(ew_v.at[j], acc_sh.at[dst_v.at[j]], add=True)
            return carry
        lax.fori_loop(0, CH, chunk, 0)
        plsc.subcore_barrier()
        _dump_partials(acc_sh, out0_hbm, out1_hbm, c, s)

    return k(d3, dst3)


def _sc_norm(src3, dst3, d3, deg0, deg1):
    @functools.partial(
        pl.kernel,
        out_type=(jax.ShapeDtypeStruct((NW, CH, C), jnp.float32),
                  jax.ShapeDtypeStruct((N_PAD,), jnp.float32)),
        mesh=_mesh(),
        compiler_params=pltpu.CompilerParams(needs_layout_passes=False, use_tc_tiling_on_sc=False),
        scratch_types=[
            pltpu.VMEM((CH, C), jnp.int32),
            pltpu.VMEM((CH, C), jnp.int32),
            pltpu.VMEM((CH, C), jnp.float32),
            pltpu.VMEM((CH, C), jnp.float32),
            pltpu.VMEM((N_PAD,), jnp.float32),
            pltpu.VMEM((N_PAD,), jnp.float32),
        ])
    def k(src_hbm, dst_hbm, d_hbm, deg0_hbm, deg1_hbm,
          nrm_hbm, invdeg_hbm,
          src_v, dst_v, ew_v, nrm_v, p0_v, p1_v):
        c = lax.axis_index("c")
        s = lax.axis_index("s")
        wid = c * NS + s
        pltpu.sync_copy(src_hbm.at[wid], src_v)
        pltpu.sync_copy(dst_hbm.at[wid], dst_v)
        pltpu.sync_copy(d_hbm.at[wid], ew_v)
        pltpu.sync_copy(deg0_hbm, p0_v)
        pltpu.sync_copy(deg1_hbm, p1_v)

        def dv(i, carry):
            sl = pl.ds(i * L, L)
            d = p0_v[sl] + p1_v[sl] + 1.0
            y = _rsqrt16(d)
            p1_v[sl] = y
            p0_v[sl] = y * y
            return carry
        lax.fori_loop(0, N_PAD // L, dv, 0)

        @pl.when(c == 0)
        def _():
            off = _off640(s)
            pltpu.sync_copy(p0_v.at[pl.ds(off, 640)],
                            invdeg_hbm.at[pl.ds(off, 640)])

        def nj(j, carry):
            for g in range(C // L):
                sl = pl.ds(g * L, L)
                nv = (plsc.load_gather(p1_v, [src_v[j, sl]])
                      * _ew16(ew_v[j, sl])
                      * plsc.load_gather(p1_v, [dst_v[j, sl]]))
                nrm_v[j, sl] = nv
            return carry
        lax.fori_loop(0, CH, nj, 0)
        pltpu.sync_copy(nrm_v, nrm_hbm.at[wid])

    return k(src3, dst3, d3, deg0, deg1)


def _sc_agg(hbf, packed3, nrm3, d):
    @functools.partial(
        pl.kernel,
        out_type=(jax.ShapeDtypeStruct((N_PAD, d), jnp.float32),
                  jax.ShapeDtypeStruct((N_PAD, d), jnp.float32)),
        mesh=_mesh(),
        compiler_params=pltpu.CompilerParams(needs_layout_passes=False, use_tc_tiling_on_sc=False),
        scratch_types=[
            pltpu.VMEM((NCHP, CC), jnp.int32),
            pltpu.VMEM((NCH, CC), jnp.float32),
            pltpu.VMEM((3, CC), jnp.int32),
            pltpu.VMEM((3, CC), jnp.int32),
            pltpu.VMEM((CC, d // 2), jnp.int32),
            pltpu.VMEM((CC, d // 2), jnp.int32),
            pltpu.VMEM((CC, d // 2), jnp.int32),
            pltpu.VMEM((CC, d), jnp.float32),
            pltpu.VMEM((CC, d), jnp.float32),
            pltpu.VMEM_SHARED((N_PAD, d), jnp.float32),
            pltpu.SemaphoreType.DMA,
            pltpu.SemaphoreType.DMA,
            pltpu.SemaphoreType.DMA,
            pltpu.SemaphoreType.DMA,
            pltpu.SemaphoreType.DMA,
        ])
    def k(h_hbm, packed_hbm, nrm_hbm, out0_hbm, out1_hbm,
          packed_v, nrm_v, srcring, dstring, rb0, rb1, rb2, rf0, rf1, acc_sh,
          g0, g1, g2, s0, s1):
        c = lax.axis_index("c")
        s = lax.axis_index("s")
        wid = c * NS + s
        pltpu.sync_copy(packed_hbm.at[wid], packed_v)
        pltpu.sync_copy(nrm_hbm.at[wid], nrm_v)
        _agg_phase(h_hbm, out0_hbm, out1_hbm, packed_v, nrm_v, srcring,
                   dstring, (rb0, rb1, rb2), (rf0, rf1), (g0, g1, g2),
                   (s0, s1), acc_sh, c, s, d)

    return k(hbf, packed3, nrm3)


def _sc_layer3(h3, src3, dst3, nrm3):
    @functools.partial(
        pl.kernel,
        out_type=(jax.ShapeDtypeStruct((N_PAD,), jnp.float32),
                  jax.ShapeDtypeStruct((N_PAD,), jnp.float32)),
        mesh=_mesh(),
        compiler_params=pltpu.CompilerParams(needs_layout_passes=False, use_tc_tiling_on_sc=False),
        scratch_types=[
            pltpu.VMEM((N_PAD,), jnp.float32),
            pltpu.VMEM((CH, C), jnp.int32),
            pltpu.VMEM((CH, C), jnp.int32),
            pltpu.VMEM((CH, C), jnp.float32),
            pltpu.VMEM((C,), jnp.float32),
            pltpu.VMEM((640,), jnp.float32),
            pltpu.VMEM_SHARED((N_PAD,), jnp.float32),
        ])
    def k(h_hbm, src_hbm, dst_hbm, nrm_hbm, out0_hbm, out1_hbm,
          tab_v, src_v, dst_v, nrm_v, val_v, zro_v, acc_sh):
        c = lax.axis_index("c")
        s = lax.axis_index("s")
        wid = c * NS + s
        pltpu.sync_copy(h_hbm, tab_v)
        pltpu.sync_copy(src_hbm.at[wid], src_v)
        pltpu.sync_copy(dst_hbm.at[wid], dst_v)
        pltpu.sync_copy(nrm_hbm.at[wid], nrm_v)

        def z(i, carry):
            zro_v[pl.ds(i * L, L)] = jnp.zeros((L,), jnp.float32)
            return carry
        lax.fori_loop(0, 640 // L, z, 0)
        pltpu.sync_copy(zro_v, acc_sh.at[pl.ds(_off640(s), 640)])
        plsc.subcore_barrier()

        def chunk(j, carry):
            for g in range(C // L):
                sl = pl.ds(g * L, L)
                v = plsc.load_gather(tab_v, [src_v[j, sl]]) * nrm_v[j, sl]
                val_v[sl] = v
            pltpu.sync_copy(val_v, acc_sh.at[dst_v.at[j]], add=True)
            return carry
        lax.fori_loop(0, CH, chunk, 0)
        plsc.subcore_barrier()
        _dump_partials(acc_sh, out0_hbm, out1_hbm, c, s)

    return k(h3, src3, dst3, nrm3)


def _half_perms(d):
    lo = np.concatenate([np.arange(k, k + 16) for k in range(0, d, 32)])
    hi = lo + 16
    mk = lambda q: jnp.asarray(
        (np.arange(d)[:, None] == q[None, :]).astype(np.float32))
    return mk(lo), mk(hi)


def _pack_rows(h, plo_ref, phi_ref):
    lo = jnp.dot(h, plo_ref[...], precision=lax.Precision.HIGHEST,
                 preferred_element_type=jnp.float32)
    hi = jnp.dot(h, phi_ref[...], precision=lax.Precision.HIGHEST,
                 preferred_element_type=jnp.float32)
    bl = lax.shift_right_logical(
        lax.bitcast_convert_type(lo.astype(jnp.bfloat16).astype(jnp.float32),
                                 jnp.int32), 16)
    bh = lax.bitwise_and(
        lax.bitcast_convert_type(hi.astype(jnp.bfloat16).astype(jnp.float32),
                                 jnp.int32), jnp.int32(-65536))
    return lax.bitwise_or(bl, bh)


def _mm_body(x_ref, w_ref, plo_ref, phi_ref, o_ref, t_ref):
    h = jnp.dot(x_ref[...], w_ref[...], precision=lax.Precision.HIGHEST,
                preferred_element_type=jnp.float32)
    o_ref[...] = h
    t_ref[...] = _pack_rows(h, plo_ref, phi_ref)


def _tc_matmul(xp, w, plo, phi):
    m, kdim = xp.shape
    n = w.shape[1]
    return pl.pallas_call(
        _mm_body,
        grid=(m // 128,),
        in_specs=[pl.BlockSpec((128, kdim), lambda i: (i, 0)),
                  pl.BlockSpec((kdim, n), lambda i: (0, 0)),
                  pl.BlockSpec((n, n // 2), lambda i: (0, 0)),
                  pl.BlockSpec((n, n // 2), lambda i: (0, 0))],
        out_specs=[pl.BlockSpec((128, n), lambda i: (i, 0)),
                   pl.BlockSpec((128, n // 2), lambda i: (i, 0))],
        out_shape=[jax.ShapeDtypeStruct((m, n), jnp.float32),
                   jax.ShapeDtypeStruct((m, n // 2), jnp.int32)],
    )(xp, w, plo, phi)


def _fuse1_body(a0_ref, a1_ref, h_ref, iv_ref, b_ref, w_ref, plo_ref,
                phi_ref, o_ref, t_ref):
    a = a0_ref[...] + a1_ref[...] + h_ref[...] * iv_ref[...] + b_ref[...]
    h2 = jnp.dot(jnp.maximum(a, 0.0), w_ref[...],
                 precision=lax.Precision.HIGHEST,
                 preferred_element_type=jnp.float32)
    o_ref[...] = h2
    t_ref[...] = _pack_rows(h2, plo_ref, phi_ref)


def _tc_fuse1(a0, a1, h1, invd, b1, w2, plo, phi):
    return pl.pallas_call(
        _fuse1_body,
        grid=(N_PAD // 128,),
        in_specs=[pl.BlockSpec((128, 128), lambda i: (i, 0)),
                  pl.BlockSpec((128, 128), lambda i: (i, 0)),
                  pl.BlockSpec((128, 128), lambda i: (i, 0)),
                  pl.BlockSpec((128, 1), lambda i: (i, 0)),
                  pl.BlockSpec((1, 128), lambda i: (0, 0)),
                  pl.BlockSpec((128, 64), lambda i: (0, 0)),
                  pl.BlockSpec((64, 32), lambda i: (0, 0)),
                  pl.BlockSpec((64, 32), lambda i: (0, 0))],
        out_specs=[pl.BlockSpec((128, 64), lambda i: (i, 0)),
                   pl.BlockSpec((128, 32), lambda i: (i, 0))],
        out_shape=[jax.ShapeDtypeStruct((N_PAD, 64), jnp.float32),
                   jax.ShapeDtypeStruct((N_PAD, 32), jnp.int32)],
    )(a0, a1, h1, invd, b1.reshape(1, 128), w2, plo, phi)


def _fuse2_body(a0_ref, a1_ref, h_ref, iv_ref, b_ref, w_ref, o_ref):
    a = a0_ref[...] + a1_ref[...] + h_ref[...] * iv_ref[...] + b_ref[...]
    o_ref[...] = jnp.sum(jnp.maximum(a, 0.0) * w_ref[...], axis=1,
                         keepdims=True)


def _tc_fuse2(a0, a1, h2, invd, b2, w3):
    return pl.pallas_call(
        _fuse2_body,
        grid=(N_PAD // 128,),
        in_specs=[pl.BlockSpec((128, 64), lambda i: (i, 0)),
                  pl.BlockSpec((128, 64), lambda i: (i, 0)),
                  pl.BlockSpec((128, 64), lambda i: (i, 0)),
                  pl.BlockSpec((128, 1), lambda i: (i, 0)),
                  pl.BlockSpec((1, 64), lambda i: (0, 0)),
                  pl.BlockSpec((1, 64), lambda i: (0, 0))],
        out_specs=pl.BlockSpec((128, 1), lambda i: (i, 0)),
        out_shape=jax.ShapeDtypeStruct((N_PAD, 1), jnp.float32),
    )(a0, a1, h2, invd, b2.reshape(1, 64), w3.reshape(1, 64))


def _fuse3_body(a0_ref, a1_ref, h_ref, iv_ref, b_ref, o_ref):
    o_ref[...] = (a0_ref[...] + a1_ref[...] + h_ref[...] * iv_ref[...]
                  + b_ref[...])


def _tc_fuse3(a0, a1, h3r, invr, b3):
    return pl.pallas_call(
        _fuse3_body,
        grid=(1,),
        in_specs=[pl.BlockSpec((CH, 128), lambda i: (0, 0)),
                  pl.BlockSpec((CH, 128), lambda i: (0, 0)),
                  pl.BlockSpec((CH, 128), lambda i: (0, 0)),
                  pl.BlockSpec((CH, 128), lambda i: (0, 0)),
                  pl.BlockSpec((1, 1), lambda i: (0, 0))],
        out_specs=pl.BlockSpec((CH, 128), lambda i: (0, 0)),
        out_shape=jax.ShapeDtypeStruct((CH, 128), jnp.float32),
    )(a0, a1, h3r, invr, b3.reshape(1, 1))


def kernel(x, edge_index, edge_attr, W1, b1, W2, b2, W3, b3):
    src = edge_index[0].astype(jnp.int32)
    dst = edge_index[1].astype(jnp.int32)
    src3 = jnp.pad(src, (0, E_PAD - E)).reshape(NW, CH, C)
    dst3 = jnp.pad(dst, (0, E_PAD - E)).reshape(NW, CH, C)
    dpad = jnp.pad(edge_attr.reshape(-1), (0, E_PAD - E),
                   constant_values=1e9)
    xp = jnp.pad(x, ((0, N_PAD - N), (0, 0)))

    d3 = dpad.reshape(NW, CH, C)
    plo128, phi128 = _half_perms(128)
    plo64, phi64 = _half_perms(64)
    h1, t1 = _tc_matmul(xp, W1, plo128, phi128)

    deg0, deg1 = _sc_deg(d3, dst3)
    nrm, invdeg = _sc_norm(src3, dst3, d3, deg0, deg1)
    invd2 = invdeg.reshape(N_PAD, 1)

    packed = src3.reshape(NW, EPW) | (dst3.reshape(NW, EPW) << 16)
    packed3 = jnp.pad(packed, ((0, 0), (0, NCHP * CC - EPW))
                      ).reshape(NW, NCHP, CC)
    nrm158 = nrm.reshape(NW, NCH, CC)

    a10, a11 = _sc_agg(t1, packed3, nrm158, 128)
    h2, t2 = _tc_fuse1(a10, a11, h1, invd2, b1, W2, plo64, phi64)
    a20, a21 = _sc_agg(t2, packed3, nrm158, 64)
    h3 = _tc_fuse2(a20, a21, h2, invd2, b2, W3)
    a30, a31 = _sc_layer3(h3.reshape(-1), src3, dst3, nrm)

    out = _tc_fuse3(a30.reshape(CH, 128), a31.reshape(CH, 128),
                    h3.reshape(CH, 128),
                    invdeg.reshape(CH, 128), b3)
    return out.reshape(-1)[:N]


# speedup vs baseline: 1.0141x; 1.0122x over previous
import functools

import jax
import jax.numpy as jnp
from jax import lax
from jax.experimental import pallas as pl
from jax.experimental.pallas import tpu as pltpu
from jax.experimental.pallas import tpu_sc as plsc

N = 10000
E = 320000
NC, NS, L = 2, 16, 16
NW = NC * NS
C = 128
CH = 79
EPW = C * CH
E_PAD = NW * EPW
N_PAD = 79 * 128
NPW = N_PAD // NS

_mesh = functools.partial(
    plsc.VectorSubcoreMesh,
    core_axis_name="c", subcore_axis_name="s",
    num_cores=NC, num_subcores=NS)


def _rsqrt16(x):
    b = lax.bitcast_convert_type(x, jnp.int32)
    i = jnp.int32(0x5F3759DF) - lax.shift_right_logical(b, 1)
    y = lax.bitcast_convert_type(i, jnp.float32)
    for _ in range(4):
        y = y * (1.5 - 0.5 * x * y * y)
    return y


def _zero_rows(row_v, d, n):
    def body(i, carry):
        for k in range(d // L):
            row_v[i, pl.ds(k * L, L)] = jnp.zeros((L,), jnp.float32)
        return carry
    lax.fori_loop(0, n, body, 0)


CC = 64
NCH = EPW // CC
NCHP = 160


def _agg_phase(h_hbm, out0_hbm, out1_hbm, packed_v, nrm_v, srcring, dstring,
               rows, gsems, ssems, acc_sh, c, s, d):
    mask16 = jnp.int32(0xFFFF)

    def unpack(jj, slot):
        for g in range(CC // L):
            sl = pl.ds(g * L, L)
            p = packed_v[jj, sl]
            srcring[slot, sl] = lax.bitwise_and(p, mask16)
            dstring[slot, sl] = lax.shift_right_logical(p, 16)

    def g_issue(slot):
        pltpu.async_copy(h_hbm.at[srcring.at[slot]], rows[slot], gsems[slot])

    def g_wait(slot):
        pltpu.make_async_copy(h_hbm.at[srcring.at[slot]], rows[slot],
                              gsems[slot]).wait()

    def s_issue(slot):
        pltpu.async_copy(rows[slot], acc_sh.at[dstring.at[slot]],
                         ssems[slot], add=True)

    def s_wait(slot):
        pltpu.make_async_copy(rows[slot], acc_sh.at[dstring.at[slot]],
                              ssems[slot]).wait()

    def scale(j, slot):
        jj = jnp.full((L,), j, dtype=jnp.int32)
        row_v = rows[slot]

        def edge8(q, ecarry):
            for t in range(8):
                e = q * 8 + t
                w = plsc.load_gather(nrm_v, [jj, jnp.full((L,), e, jnp.int32)])
                for k in range(d // L):
                    sl = pl.ds(k * L, L)
                    row_v[e, sl] = row_v[e, sl] * w
            return ecarry
        lax.fori_loop(0, CC // 8, edge8, 0)

    def turn(j, b):
        bp = (b + 2) % 3
        g_wait(b)
        scale(j, b)
        s_issue(b)

        @pl.when(j >= 1)
        def _():
            s_wait(bp)
        unpack(j + 2, bp)

        @pl.when(j + 2 < NCH)
        def _():
            g_issue(bp)

    unpack(jnp.int32(0), 0)
    unpack(jnp.int32(1), 1)
    g_issue(0)
    g_issue(1)

    _zero_rows(rows[2], d, CC)
    base = s * NPW
    for t in range(NPW // CC):
        pltpu.sync_copy(rows[2], acc_sh.at[pl.ds(base + t * CC, CC)])
    rem = NPW % CC
    if rem:
        pltpu.sync_copy(rows[2].at[pl.ds(0, rem)],
                        acc_sh.at[pl.ds(base + (NPW // CC) * CC, rem)])
    plsc.subcore_barrier()

    def body(q, carry):
        j = q * 3
        turn(j, 0)
        turn(j + 1, 1)
        turn(j + 2, 2)
        return carry
    lax.fori_loop(0, NCH // 3, body, 0)
    turn(jnp.int32(NCH - 2), (NCH - 2) % 3)
    turn(jnp.int32(NCH - 1), (NCH - 1) % 3)
    s_wait((NCH - 1) % 3)

    plsc.subcore_barrier()

    @pl.when(c == 0)
    def _():
        pltpu.sync_copy(acc_sh.at[pl.ds(s * NPW, NPW)],
                        out0_hbm.at[pl.ds(s * NPW, NPW)])

    @pl.when(c == 1)
    def _():
        pltpu.sync_copy(acc_sh.at[pl.ds(s * NPW, NPW)],
                        out1_hbm.at[pl.ds(s * NPW, NPW)])


def _off640(s):
    return jnp.minimum(s * 640, N_PAD - 640)


def _dump_partials(acc_sh, out0_hbm, out1_hbm, c, s):
    off = _off640(s)

    @pl.when(c == 0)
    def _():
        pltpu.sync_copy(acc_sh.at[pl.ds(off, 640)],
                        out0_hbm.at[pl.ds(off, 640)])

    @pl.when(c == 1)
    def _():
        pltpu.sync_copy(acc_sh.at[pl.ds(off, 640)],
                        out1_hbm.at[pl.ds(off, 640)])


def _sc_deg(ew3, dst3):
    @functools.partial(
        pl.kernel,
        out_type=(jax.ShapeDtypeStruct((N_PAD,), jnp.float32),
                  jax.ShapeDtypeStruct((N_PAD,), jnp.float32)),
        mesh=_mesh(),
        compiler_params=pltpu.CompilerParams(needs_layout_passes=False, use_tc_tiling_on_sc=False),
        scratch_types=[
            pltpu.VMEM((CH, C), jnp.float32),
            pltpu.VMEM((CH, C), jnp.int32),
            pltpu.VMEM((640,), jnp.float32),
            pltpu.VMEM_SHARED((N_PAD,), jnp.float32),
        ])
    def k(ew_hbm, dst_hbm, out0_hbm, out1_hbm, ew_v, dst_v, zro_v, acc_sh):
        c = lax.axis_index("c")
        s = lax.axis_index("s")
        wid = c * NS + s
        pltpu.sync_copy(ew_hbm.at[wid], ew_v)
        pltpu.sync_copy(dst_hbm.at[wid], dst_v)

        def z(i, carry):
            zro_v[pl.ds(i * L, L)] = jnp.zeros((L,), jnp.float32)
            return carry
        lax.fori_loop(0, 640 // L, z, 0)
        pltpu.sync_copy(zro_v, acc_sh.at[pl.ds(_off640(s), 640)])
        plsc.subcore_barrier()

        def chunk(j, carry):
            pltpu.sync_copy(ew_v.at[j], acc_sh.at[dst_v.at[j]], add=True)
            return carry
        lax.fori_loop(0, CH, chunk, 0)
        plsc.subcore_barrier()
        _dump_partials(acc_sh, out0_hbm, out1_hbm, c, s)

    return k(ew3, dst3)


def _sc_norm(src3, dst3, ew3, deg0, deg1):
    @functools.partial(
        pl.kernel,
        out_type=(jax.ShapeDtypeStruct((NW, CH, C), jnp.float32),
                  jax.ShapeDtypeStruct((N_PAD,), jnp.float32)),
        mesh=_mesh(),
        compiler_params=pltpu.CompilerParams(needs_layout_passes=False, use_tc_tiling_on_sc=False),
        scratch_types=[
            pltpu.VMEM((CH, C), jnp.int32),
            pltpu.VMEM((CH, C), jnp.int32),
            pltpu.VMEM((CH, C), jnp.float32),
            pltpu.VMEM((CH, C), jnp.float32),
            pltpu.VMEM((N_PAD,), jnp.float32),
            pltpu.VMEM((N_PAD,), jnp.float32),
        ])
    def k(src_hbm, dst_hbm, ew_hbm, deg0_hbm, deg1_hbm,
          nrm_hbm, invdeg_hbm,
          src_v, dst_v, ew_v, nrm_v, p0_v, p1_v):
        c = lax.axis_index("c")
        s = lax.axis_index("s")
        wid = c * NS + s
        pltpu.sync_copy(src_hbm.at[wid], src_v)
        pltpu.sync_copy(dst_hbm.at[wid], dst_v)
        pltpu.sync_copy(ew_hbm.at[wid], ew_v)
        pltpu.sync_copy(deg0_hbm, p0_v)
        pltpu.sync_copy(deg1_hbm, p1_v)

        def dv(i, carry):
            sl = pl.ds(i * L, L)
            d = p0_v[sl] + p1_v[sl] + 1.0
            y = _rsqrt16(d)
            p1_v[sl] = y
            p0_v[sl] = y * y
            return carry
        lax.fori_loop(0, N_PAD // L, dv, 0)

        @pl.when(c == 0)
        def _():
            off = _off640(s)
            pltpu.sync_copy(p0_v.at[pl.ds(off, 640)],
                            invdeg_hbm.at[pl.ds(off, 640)])

        def nj(j, carry):
            for g in range(C // L):
                sl = pl.ds(g * L, L)
                nv = (plsc.load_gather(p1_v, [src_v[j, sl]])
                      * ew_v[j, sl]
                      * plsc.load_gather(p1_v, [dst_v[j, sl]]))
                nrm_v[j, sl] = nv
            return carry
        lax.fori_loop(0, CH, nj, 0)
        pltpu.sync_copy(nrm_v, nrm_hbm.at[wid])

    return k(src3, dst3, ew3, deg0, deg1)


def _sc_agg(h, packed3, nrm3, d):
    @functools.partial(
        pl.kernel,
        out_type=(jax.ShapeDtypeStruct((N_PAD, d), jnp.float32),
                  jax.ShapeDtypeStruct((N_PAD, d), jnp.float32)),
        mesh=_mesh(),
        compiler_params=pltpu.CompilerParams(needs_layout_passes=False, use_tc_tiling_on_sc=False),
        scratch_types=[
            pltpu.VMEM((NCHP, CC), jnp.int32),
            pltpu.VMEM((NCH, CC), jnp.float32),
            pltpu.VMEM((3, CC), jnp.int32),
            pltpu.VMEM((3, CC), jnp.int32),
            pltpu.VMEM((CC, d), jnp.float32),
            pltpu.VMEM((CC, d), jnp.float32),
            pltpu.VMEM((CC, d), jnp.float32),
            pltpu.VMEM_SHARED((N_PAD, d), jnp.float32),
            pltpu.SemaphoreType.DMA,
            pltpu.SemaphoreType.DMA,
            pltpu.SemaphoreType.DMA,
            pltpu.SemaphoreType.DMA,
            pltpu.SemaphoreType.DMA,
            pltpu.SemaphoreType.DMA,
        ])
    def k(h_hbm, packed_hbm, nrm_hbm, out0_hbm, out1_hbm,
          packed_v, nrm_v, srcring, dstring, row0, row1, row2, acc_sh,
          g0, g1, g2, s0, s1, s2):
        c = lax.axis_index("c")
        s = lax.axis_index("s")
        wid = c * NS + s
        pltpu.sync_copy(packed_hbm.at[wid], packed_v)
        pltpu.sync_copy(nrm_hbm.at[wid], nrm_v)
        _agg_phase(h_hbm, out0_hbm, out1_hbm, packed_v, nrm_v, srcring,
                   dstring, (row0, row1, row2), (g0, g1, g2), (s0, s1, s2),
                   acc_sh, c, s, d)

    return k(h, packed3, nrm3)


def _sc_layer3(h3, src3, dst3, nrm3):
    @functools.partial(
        pl.kernel,
        out_type=(jax.ShapeDtypeStruct((N_PAD,), jnp.float32),
                  jax.ShapeDtypeStruct((N_PAD,), jnp.float32)),
        mesh=_mesh(),
        compiler_params=pltpu.CompilerParams(needs_layout_passes=False, use_tc_tiling_on_sc=False),
        scratch_types=[
            pltpu.VMEM((N_PAD,), jnp.float32),
            pltpu.VMEM((CH, C), jnp.int32),
            pltpu.VMEM((CH, C), jnp.int32),
            pltpu.VMEM((CH, C), jnp.float32),
            pltpu.VMEM((C,), jnp.float32),
            pltpu.VMEM((640,), jnp.float32),
            pltpu.VMEM_SHARED((N_PAD,), jnp.float32),
        ])
    def k(h_hbm, src_hbm, dst_hbm, nrm_hbm, out0_hbm, out1_hbm,
          tab_v, src_v, dst_v, nrm_v, val_v, zro_v, acc_sh):
        c = lax.axis_index("c")
        s = lax.axis_index("s")
        wid = c * NS + s
        pltpu.sync_copy(h_hbm, tab_v)
        pltpu.sync_copy(src_hbm.at[wid], src_v)
        pltpu.sync_copy(dst_hbm.at[wid], dst_v)
        pltpu.sync_copy(nrm_hbm.at[wid], nrm_v)

        def z(i, carry):
            zro_v[pl.ds(i * L, L)] = jnp.zeros((L,), jnp.float32)
            return carry
        lax.fori_loop(0, 640 // L, z, 0)
        pltpu.sync_copy(zro_v, acc_sh.at[pl.ds(_off640(s), 640)])
        plsc.subcore_barrier()

        def chunk(j, carry):
            for g in range(C // L):
                sl = pl.ds(g * L, L)
                v = plsc.load_gather(tab_v, [src_v[j, sl]]) * nrm_v[j, sl]
                val_v[sl] = v
            pltpu.sync_copy(val_v, acc_sh.at[dst_v.at[j]], add=True)
            return carry
        lax.fori_loop(0, CH, chunk, 0)
        plsc.subcore_barrier()
        _dump_partials(acc_sh, out0_hbm, out1_hbm, c, s)

    return k(h3, src3, dst3, nrm3)


def _ew_body(d_ref, o_ref):
    d = d_ref[...] * (1.0 / 200.0)
    o_ref[...] = jnp.exp(-(d * d))


def _tc_ew(d2):
    return pl.pallas_call(
        _ew_body,
        grid=(4,),
        in_specs=[pl.BlockSpec((E_PAD // 128 // 4, 128), lambda i: (i, 0))],
        out_specs=pl.BlockSpec((E_PAD // 128 // 4, 128), lambda i: (i, 0)),
        out_shape=jax.ShapeDtypeStruct((E_PAD // 128, 128), jnp.float32),
    )(d2)


def _mm_body(x_ref, w_ref, o_ref):
    o_ref[...] = jnp.dot(x_ref[...], w_ref[...],
                         precision=lax.Precision.HIGHEST,
                         preferred_element_type=jnp.float32)


def _tc_matmul(xp, w):
    m, kdim = xp.shape
    n = w.shape[1]
    return pl.pallas_call(
        _mm_body,
        grid=(m // 128,),
        in_specs=[pl.BlockSpec((128, kdim), lambda i: (i, 0)),
                  pl.BlockSpec((kdim, n), lambda i: (0, 0))],
        out_specs=pl.BlockSpec((128, n), lambda i: (i, 0)),
        out_shape=jax.ShapeDtypeStruct((m, n), jnp.float32),
    )(xp, w)


def _fuse1_body(a0_ref, a1_ref, h_ref, iv_ref, b_ref, w_ref, o_ref):
    a = a0_ref[...] + a1_ref[...] + h_ref[...] * iv_ref[...] + b_ref[...]
    o_ref[...] = jnp.dot(jnp.maximum(a, 0.0), w_ref[...],
                         precision=lax.Precision.HIGHEST,
                         preferred_element_type=jnp.float32)


def _tc_fuse1(a0, a1, h1, invd, b1, w2):
    return pl.pallas_call(
        _fuse1_body,
        grid=(N_PAD // 128,),
        in_specs=[pl.BlockSpec((128, 128), lambda i: (i, 0)),
                  pl.BlockSpec((128, 128), lambda i: (i, 0)),
                  pl.BlockSpec((128, 128), lambda i: (i, 0)),
                  pl.BlockSpec((128, 1), lambda i: (i, 0)),
                  pl.BlockSpec((1, 128), lambda i: (0, 0)),
                  pl.BlockSpec((128, 64), lambda i: (0, 0))],
        out_specs=pl.BlockSpec((128, 64), lambda i: (i, 0)),
        out_shape=jax.ShapeDtypeStruct((N_PAD, 64), jnp.float32),
    )(a0, a1, h1, invd, b1.reshape(1, 128), w2)


def _fuse2_body(a0_ref, a1_ref, h_ref, iv_ref, b_ref, w_ref, o_ref):
    a = a0_ref[...] + a1_ref[...] + h_ref[...] * iv_ref[...] + b_ref[...]
    o_ref[...] = jnp.sum(jnp.maximum(a, 0.0) * w_ref[...], axis=1,
                         keepdims=True)


def _tc_fuse2(a0, a1, h2, invd, b2, w3):
    return pl.pallas_call(
        _fuse2_body,
        grid=(N_PAD // 128,),
        in_specs=[pl.BlockSpec((128, 64), lambda i: (i, 0)),
                  pl.BlockSpec((128, 64), lambda i: (i, 0)),
                  pl.BlockSpec((128, 64), lambda i: (i, 0)),
                  pl.BlockSpec((128, 1), lambda i: (i, 0)),
                  pl.BlockSpec((1, 64), lambda i: (0, 0)),
                  pl.BlockSpec((1, 64), lambda i: (0, 0))],
        out_specs=pl.BlockSpec((128, 1), lambda i: (i, 0)),
        out_shape=jax.ShapeDtypeStruct((N_PAD, 1), jnp.float32),
    )(a0, a1, h2, invd, b2.reshape(1, 64), w3.reshape(1, 64))


def _fuse3_body(a0_ref, a1_ref, h_ref, iv_ref, b_ref, o_ref):
    o_ref[...] = (a0_ref[...] + a1_ref[...] + h_ref[...] * iv_ref[...]
                  + b_ref[...])


def _tc_fuse3(a0, a1, h3r, invr, b3):
    return pl.pallas_call(
        _fuse3_body,
        grid=(1,),
        in_specs=[pl.BlockSpec((CH, 128), lambda i: (0, 0)),
                  pl.BlockSpec((CH, 128), lambda i: (0, 0)),
                  pl.BlockSpec((CH, 128), lambda i: (0, 0)),
                  pl.BlockSpec((CH, 128), lambda i: (0, 0)),
                  pl.BlockSpec((1, 1), lambda i: (0, 0))],
        out_specs=pl.BlockSpec((CH, 128), lambda i: (0, 0)),
        out_shape=jax.ShapeDtypeStruct((CH, 128), jnp.float32),
    )(a0, a1, h3r, invr, b3.reshape(1, 1))


def kernel(x, edge_index, edge_attr, W1, b1, W2, b2, W3, b3):
    src = edge_index[0].astype(jnp.int32)
    dst = edge_index[1].astype(jnp.int32)
    src3 = jnp.pad(src, (0, E_PAD - E)).reshape(NW, CH, C)
    dst3 = jnp.pad(dst, (0, E_PAD - E)).reshape(NW, CH, C)
    dpad = jnp.pad(edge_attr.reshape(-1), (0, E_PAD - E),
                   constant_values=1e9)
    xp = jnp.pad(x, ((0, N_PAD - N), (0, 0)))

    ew2 = _tc_ew(dpad.reshape(E_PAD // 128, 128))
    ew3 = ew2.reshape(NW, CH, C)
    h1 = _tc_matmul(xp, W1)

    deg0, deg1 = _sc_deg(ew3, dst3)
    nrm, invdeg = _sc_norm(src3, dst3, ew3, deg0, deg1)
    invd2 = invdeg.reshape(N_PAD, 1)

    packed = src3.reshape(NW, EPW) | (dst3.reshape(NW, EPW) << 16)
    packed3 = jnp.pad(packed, ((0, 0), (0, NCHP * CC - EPW))
                      ).reshape(NW, NCHP, CC)
    nrm158 = nrm.reshape(NW, NCH, CC)

    a10, a11 = _sc_agg(h1, packed3, nrm158, 128)
    h2 = _tc_fuse1(a10, a11, h1, invd2, b1, W2)
    a20, a21 = _sc_agg(h2, packed3, nrm158, 64)
    h3 = _tc_fuse2(a20, a21, h2, invd2, b2, W3)
    a30, a31 = _sc_layer3(h3.reshape(-1), src3, dst3, nrm)

    out = _tc_fuse3(a30.reshape(CH, 128), a31.reshape(CH, 128),
                    h3.reshape(CH, 128),
                    invdeg.reshape(CH, 128), b3)
    return out.reshape(-1)[:N]
